# baseline (speedup 1.0000x reference)
import functools

import jax
import jax.numpy as jnp
from jax import lax
from jax.experimental import pallas as pl
from jax.experimental.pallas import tpu as pltpu
from jax.experimental.pallas import tpu_sc as plsc

EMBED = 64
IMG_TOK = 151669
BATCH = 16
SEQ = 2048
TOT = BATCH * SEQ
NPAIR = 100000

NC, NS = 2, 16
NW = NC * NS
ROWS_PER_W = TOT // NW
CS = 128
NCH = ROWS_PER_W // CS
L = 16

RB = 800


def _pair_body(a_ref, b_ref, out_ref):
    out_ref[...] = jnp.concatenate([a_ref[...], b_ref[...]], axis=1)


def _pack_pairs_tc(table):
    return pl.pallas_call(
        _pair_body,
        grid=(NPAIR // RB,),
        in_specs=[
            pl.BlockSpec((RB, EMBED), lambda i: (i, 0)),
            pl.BlockSpec((RB, EMBED), lambda i: (i + NPAIR // RB, 0)),
        ],
        out_specs=pl.BlockSpec((RB, 2 * EMBED), lambda i: (i, 0)),
        out_shape=jax.ShapeDtypeStruct((NPAIR, 2 * EMBED), jnp.float32),
    )(table, table)


@functools.cache
def _build_sc_gather():
    mesh = plsc.VectorSubcoreMesh(
        core_axis_name="c", subcore_axis_name="s", num_cores=NC, num_subcores=NS
    )

    @functools.partial(
        pl.kernel,
        mesh=mesh,
        out_type=jax.ShapeDtypeStruct((TOT, EMBED), jnp.float32),
        scratch_types=[
            pltpu.VMEM((8, 128), jnp.int32),
            pltpu.VMEM((8, 128), jnp.int32),
            pltpu.VMEM((8, 128), jnp.int32),
            pltpu.VMEM((CS, 128), jnp.float32),
            pltpu.VMEM((CS, 128), jnp.float32),
            pltpu.VMEM((CS, EMBED), jnp.float32),
            pltpu.VMEM((CS, EMBED), jnp.float32),
            pltpu.SemaphoreType.DMA,
            pltpu.SemaphoreType.DMA,
        ],
        compiler_params=pltpu.CompilerParams(needs_layout_passes=False),
    )
    def _sc_gather(
        table2, ids2d, out_hbm, ids_v, pidx_v, half_v, dstA, dstB, rowA, rowB,
        semA, semB,
    ):
        wid = lax.axis_index("s") * NC + lax.axis_index("c")
        pltpu.sync_copy(ids2d.at[pl.ds(wid * 8, 8)], ids_v)

        def prep(i, _):
            r, c = i // 8, (i % 8) * L
            v = ids_v[r, pl.ds(c, L)]
            hi = (v >= NPAIR).astype(jnp.int32)
            pidx_v[r, pl.ds(c, L)] = v - hi * NPAIR
            half_v[r, pl.ds(c, L)] = hi * EMBED
            return 0

        lax.fori_loop(0, (8 * 128) // L, prep, 0)

        def fire(g, dst, sem):
            pltpu.async_copy(table2.at[pidx_v.at[g]], dst, sem)

        def drain(dst, sem):
            pltpu.make_async_copy(table2.at[pl.ds(0, CS)], dst, sem).wait()

        iota = lax.iota(jnp.int32, L)

        def extract(g, dst, row):
            for h in range(CS // L):
                off = half_v[g, pl.ds(h * L, L)]
                r_ix = iota + (h * L)

                def col(c, _):
                    for u in range(4):
                        c_ix = jnp.zeros((L,), jnp.int32) + (c * 4 + u)
                        v = plsc.load_gather(dst, [r_ix, off + c_ix])
                        plsc.store_scatter(row, [r_ix, c_ix], v)
                    return 0

                lax.fori_loop(0, EMBED // 4, col, 0)

        def flush(g, row):
            pltpu.sync_copy(
                row, out_hbm.at[pl.ds(wid * ROWS_PER_W + g * CS, CS)]
            )

        fire(0, dstA, semA)
        fire(1, dstB, semB)

        def step(h, _):
            g0 = 2 * h
            drain(dstA, semA)
            extract(g0, dstA, rowA)
            flush(g0, rowA)

            @pl.when(g0 + 2 < NCH)
            def _():
                fire(g0 + 2, dstA, semA)

            drain(dstB, semB)
            extract(g0 + 1, dstB, rowB)
            flush(g0 + 1, rowB)

            @pl.when(g0 + 3 < NCH)
            def _():
                fire(g0 + 3, dstB, semB)

            return 0

        lax.fori_loop(0, NCH // 2, step, 0)

    return _sc_gather


def _mask_body(ids_ref, attn_ref, am_out, im_out):
    am_out[...] = attn_ref[...] == 1
    im_out[...] = ids_ref[...] == IMG_TOK


def _masks_tc(input_ids, attention_mask):
    return pl.pallas_call(
        _mask_body,
        out_shape=(
            jax.ShapeDtypeStruct((BATCH, SEQ), jnp.bool_),
            jax.ShapeDtypeStruct((BATCH, SEQ), jnp.bool_),
        ),
    )(input_ids, attention_mask)


def kernel(pixel_values, input_ids, attention_mask, text_proj_weight):
    del pixel_values
    ids32 = input_ids.astype(jnp.int32)
    ids2d = ids32.reshape(TOT // 128, 128)
    table2 = _pack_pairs_tc(text_proj_weight)
    flat = _build_sc_gather()(table2, ids2d)
    hidden_states = flat.reshape(BATCH, SEQ, EMBED)
    attn_mask, image_mask = _masks_tc(ids32, attention_mask.astype(jnp.int32))
    return (hidden_states, attn_mask, image_mask)


# speedup vs baseline: 1.1721x; 1.1721x over previous
import functools

import jax
import jax.numpy as jnp
from jax import lax
from jax.experimental import pallas as pl
from jax.experimental.pallas import tpu as pltpu
from jax.experimental.pallas import tpu_sc as plsc

EMBED = 64
IMG_TOK = 151669
BATCH = 16
SEQ = 2048
TOT = BATCH * SEQ
NPAIR = 100000

NC, NS = 2, 16
NW = NC * NS
ROWS_PER_W = TOT // NW
CS = 64
NCH = ROWS_PER_W // CS
NBUF = 12
L = 16


@functools.cache
def _build_sc_gather():
    mesh = plsc.VectorSubcoreMesh(
        core_axis_name="c", subcore_axis_name="s", num_cores=NC, num_subcores=NS
    )

    @functools.partial(
        pl.kernel,
        mesh=mesh,
        out_type=jax.ShapeDtypeStruct((TOT, EMBED), jnp.float32),
        scratch_types=[
            pltpu.VMEM((8, 128), jnp.int32),
            pltpu.VMEM((8, 128), jnp.int32),
            [pltpu.VMEM((CS, 128), jnp.float32) for _ in range(NBUF)],
            pltpu.VMEM((CS, EMBED), jnp.float32),
            [pltpu.SemaphoreType.DMA for _ in range(NBUF)],
        ],
        compiler_params=pltpu.CompilerParams(needs_layout_passes=False),
    )
    def _sc_gather(
        table2, ids2d, out_hbm, pidx_v, half_v, dsts, rowbuf, sems
    ):
        wid = lax.axis_index("s") * NC + lax.axis_index("c")
        pltpu.sync_copy(ids2d.at[pl.ds(wid * 8, 8)], pidx_v)

        def prep(i, _):
            r, c = i // 8, (i % 8) * L
            v = pidx_v[r, pl.ds(c, L)]
            pidx_v[r, pl.ds(c, L)] = lax.shift_right_logical(v, 1)
            half_v[r, pl.ds(c, L)] = lax.bitwise_and(v, 1) * EMBED
            return 0

        lax.fori_loop(0, (8 * 128) // L, prep, 0)

        def fire(g, b):
            idx_ref = pidx_v.at[g // 2, pl.ds((g % 2) * CS, CS)]
            pltpu.async_copy(table2.at[idx_ref], dsts[b], sems[b])

        def drain(b):
            pltpu.make_async_copy(table2.at[pl.ds(0, CS)], dsts[b], sems[b]).wait()

        iota = lax.iota(jnp.int32, L)

        def extract(g, b):
            for h in range(CS // L):
                off = half_v[g // 2, pl.ds((g % 2) * CS + h * L, L)]
                r_ix = iota + (h * L)

                def col(c, _):
                    for u in range(4):
                        c_ix = jnp.zeros((L,), jnp.int32) + (c * 4 + u)
                        v = plsc.load_gather(dsts[b], [r_ix, off + c_ix])
                        plsc.store_scatter(rowbuf, [r_ix, c_ix], v)
                    return 0

                lax.fori_loop(0, EMBED // 4, col, 0)

        def flush(g):
            pltpu.sync_copy(
                rowbuf, out_hbm.at[pl.ds(wid * ROWS_PER_W + g * CS, CS)]
            )

        for g in range(NBUF):
            fire(g, g)
        for g in range(NCH):
            b = g % NBUF
            drain(b)
            extract(g, b)
            flush(g)
            if g + NBUF < NCH:
                fire(g + NBUF, b)

    return _sc_gather


def _mask_body(ids_ref, attn_ref, am_out, im_out):
    am_out[...] = attn_ref[...] == 1
    im_out[...] = ids_ref[...] == IMG_TOK


def _masks_tc(input_ids, attention_mask):
    return pl.pallas_call(
        _mask_body,
        out_shape=(
            jax.ShapeDtypeStruct((BATCH, SEQ), jnp.bool_),
            jax.ShapeDtypeStruct((BATCH, SEQ), jnp.bool_),
        ),
    )(input_ids, attention_mask)


def kernel(pixel_values, input_ids, attention_mask, text_proj_weight):
    del pixel_values
    ids32 = input_ids.astype(jnp.int32)
    ids2d = ids32.reshape(TOT // 128, 128)
    table2 = text_proj_weight.reshape(NPAIR, 2 * EMBED)
    flat = _build_sc_gather()(table2, ids2d)
    hidden_states = flat.reshape(BATCH, SEQ, EMBED)
    attn_mask, image_mask = _masks_tc(ids32, attention_mask.astype(jnp.int32))
    return (hidden_states, attn_mask, image_mask)
